# baseline (speedup 1.0000x reference)
import functools

import numpy as np
import jax
import jax.numpy as jnp
from jax import lax
from jax.experimental import pallas as pl
from jax.experimental.pallas import tpu as pltpu

_BH = 4
_BF = jnp.bfloat16
_F32 = jnp.float32


def _taps(H, W):
    ts = []
    for dy in range(3):
        for dx in range(3):
            oy, ox = dy - 1, dx - 1
            if (H == 1 and oy != 0) or (W == 1 and ox != 0):
                continue
            ts.append((oy, ox))
    return ts


def _interp1d(n_in, n_out):
    M = np.zeros((n_out, n_in), np.float64)
    if n_in == 1:
        M[:, 0] = 1.0
        return M
    scale = (n_in - 1) / (n_out - 1)
    for i in range(n_out):
        src = i * scale
        lo = min(int(np.floor(src)), n_in - 1)
        hi = min(lo + 1, n_in - 1)
        M[i, lo] += 1.0 - (src - lo)
        M[i, hi] += src - lo
    return M


def _upmat_hilo(Bh, H, W):
    U = np.kron(_interp1d(H, 2 * H), _interp1d(W, 2 * W))
    U = np.kron(np.eye(Bh), U).astype(np.float32)
    hi = U.astype(jnp.bfloat16.dtype)
    lo = (U - hi.astype(np.float32)).astype(jnp.bfloat16.dtype)
    return jnp.asarray(hi), jnp.asarray(lo)


def _poolmat(Bh, H, W):
    Ho, Wo = H // 2, W // 2
    S = np.zeros((Bh * Ho * Wo, Bh * H * W), np.float32)
    q = np.arange(Bh * Ho * Wo)
    b = q // (Ho * Wo)
    r = q % (Ho * Wo)
    S[q, b * H * W + 2 * (r // Wo) * W + 2 * (r % Wo)] = 1.0
    return jnp.asarray(S.astype(jnp.bfloat16.dtype))


def _shift(x, s):
    P = x.shape[0]
    s = s % P
    if s == 0:
        return x
    return jnp.concatenate([x[s:], x[:s]], axis=0)


def _split(v):
    hi = v.astype(_BF)
    lo = (v - hi.astype(_F32)).astype(_BF)
    return hi, lo


def _mk_masks(Bh, H, W):
    P = Bh * H * W
    if H == 1 and W == 1:
        return {}
    p = lax.broadcasted_iota(jnp.int32, (P, 1), 0)
    h = (p // W) % H
    w = p % W
    one = jnp.full((P, 1), 1.0, _F32)
    zero = jnp.zeros((P, 1), _F32)

    def cond1(v, o, n):
        if o == 0:
            return None
        c = (v >= 1) if o < 0 else (v <= n - 2)
        return jnp.where(c, one, zero).astype(_BF)

    out = {}
    for oy, ox in _taps(H, W):
        if (oy, ox) == (0, 0):
            continue
        mh, mw = cond1(h, oy, H), cond1(w, ox, W)
        m = mh if mw is None else (mw if mh is None else mh * mw)
        out[(oy, ox)] = m
    return out


def _row_onehot(P, R, row_of_p, cols_of=False):
    pi = lax.broadcasted_iota(jnp.int32, (P, R), 0)
    ri = lax.broadcasted_iota(jnp.int32, (P, R), 1)
    one = jnp.full((P, R), 1.0, _F32)
    zero = jnp.zeros((P, R), _F32)
    if cols_of:
        W_ = R // P
        cond = (ri // W_) == pi
    else:
        cond = ri == row_of_p(pi)
    return jnp.where(cond, one, zero).astype(_BF)


def _gather_cols(P, Wl, lane_of_p):
    pi = lax.broadcasted_iota(jnp.int32, (P, Wl), 0)
    li = lax.broadcasted_iota(jnp.int32, (P, Wl), 1)
    one = jnp.full((P, Wl), 1.0, _F32)
    zero = jnp.zeros((P, Wl), _F32)
    return jnp.where(li == lane_of_p(pi), one, zero)


def _conv(xs_w, sref, bref, H, W, masks):
    taps = _taps(H, W)
    acc = None
    for xv, wr in xs_w:
        xh, xl = _split(xv)
        if len(taps) == 1:
            xch, xcl = xh, xl
            wf = wr[...] if len(wr.shape) == 2 else wr[0]
        else:
            cols_h, cols_l = [], []
            for oy, ox in taps:
                sh = _shift(xh, oy * W + ox)
                sl = _shift(xl, oy * W + ox)
                if (oy, ox) != (0, 0):
                    m = masks[(oy, ox)]
                    sh = sh * m
                    sl = sl * m
                cols_h.append(sh)
                cols_l.append(sl)
            xch = jnp.concatenate(cols_h, axis=1)
            xcl = jnp.concatenate(cols_l, axis=1)
            w3 = wr[...]
            wf = w3.reshape(w3.shape[0] * w3.shape[1], w3.shape[2])
        wh, wl = _split(wf)
        d = jnp.dot(xch, wh, preferred_element_type=_F32)
        d = d + jnp.dot(xch, wl, preferred_element_type=_F32)
        d = d + jnp.dot(xcl, wh, preferred_element_type=_F32)
        acc = d if acc is None else acc + d
    y = acc * sref[...] + bref[...]
    return jnp.maximum(y, 0.0)


def _pool(x, selref, W):
    m = jnp.maximum(x, _shift(x, 1))
    m = jnp.maximum(m, _shift(m, W))
    mh, ml = _split(m)
    sel = selref[...]
    return (jnp.dot(sel, mh, preferred_element_type=_F32)
            + jnp.dot(sel, ml, preferred_element_type=_F32))


def _unet_body(*refs, treedef, hw, L, n_flat):
    x_ref = refs[0]
    p = jax.tree_util.tree_unflatten(treedef, refs[1:1 + n_flat])
    out_ref = refs[1 + n_flat]
    bufs = refs[2 + n_flat:-1]
    sems = refs[-1]
    masks = [_mk_masks(_BH, *hw[l]) for l in range(L + 1)]

    srcs = []
    for i in (1, 2, 3):
        w1 = p["downs"][i][0][0]
        w2 = p["downs"][i][3]
        if i == 3:
            srcs += [w1.at[4], w2.at[4]]
        else:
            srcs += [w1, w2]
    for i in range(L):
        (a, b), _, _, w2, _, _ = p["ups"][i]
        srcs += [a, b, w2]
    copies = [pltpu.make_async_copy(src, buf, sems.at[k])
              for k, (src, buf) in enumerate(zip(srcs, bufs))]
    for c in copies:
        c.start()

    def wait(lo, hi):
        for c in copies[lo:hi]:
            c.wait()

    def dconv(xs_w1, cp, lvl):
        _, s1, b1, w2, s2, b2 = cp
        H, W = hw[lvl]
        h = _conv(xs_w1, s1, b1, H, W, masks[lvl])
        return _conv([(h, w2)], s2, b2, H, W, masks[lvl])

    BH_, Wl = x_ref.shape
    P0 = BH_ * Wl
    x2d = x_ref[...]
    sel_rows = _row_onehot(P0, BH_, lambda pi: pi // Wl)
    x2h, x2l = _split(x2d)
    y1 = jnp.dot(sel_rows, x2h, preferred_element_type=_F32)
    y1 = y1 + jnp.dot(sel_rows, x2l, preferred_element_type=_F32)
    lane_m = _gather_cols(P0, Wl, lambda pi: pi % Wl)
    x_col = jnp.sum(y1 * lane_m, axis=1, keepdims=True)

    cur = dconv([(x_col, p["inc"][0][0])], p["inc"], 0)
    skips = [cur]
    for i in range(L):
        pooled = _pool(cur, p["pool"][i], hw[i][1])
        cp = p["downs"][i]
        if i >= 1:
            wait(2 * (i - 1), 2 * i)
            cp = ((bufs[2 * (i - 1)],), cp[1], cp[2],
                  bufs[2 * i - 1], cp[4], cp[5])
        cur = dconv([(pooled, cp[0][0])], cp, i + 1)
        skips.append(cur)

    for i in range(L):
        uh, ul = p["upmat"][i]
        ch, cl = _split(cur)
        up = jnp.dot(uh[...], ch, preferred_element_type=_F32)
        up = up + jnp.dot(uh[...], cl, preferred_element_type=_F32)
        up = up + jnp.dot(ul[...], ch, preferred_element_type=_F32)
        wait(6 + 3 * i, 9 + 3 * i)
        w1a, w1b, w2 = bufs[6 + 3 * i], bufs[7 + 3 * i], bufs[8 + 3 * i]
        cp0 = p["ups"][i]
        cp = ((w1a, w1b), cp0[1], cp0[2], w2, cp0[4], cp0[5])
        cur = dconv([(skips[L - 1 - i], w1a), (up, w1b)],
                    cp, L - 1 - i)

    wo, bo = p["outc"]
    y = jnp.sum(cur * wo[...], axis=1, keepdims=True) + bo[...]
    y2 = y * lane_m
    sel_back = _row_onehot(BH_, P0, lambda ri: ri // 1, cols_of=True)
    y2h, y2l = _split(y2)
    o = jnp.dot(sel_back, y2h, preferred_element_type=_F32)
    o = o + jnp.dot(sel_back, y2l, preferred_element_type=_F32)
    out_ref[...] = o


def kernel(inp, p0, p1, p2, p3, p4, p5, p6, p7, p8, p9, p10, p11, p12, p13,
           p14, p15, p16, p17, p18, p19, p20, p21, p22, p23, p24, p25, p26,
           p27, p28, p29, p30, p31, p32, p33, p34, p35, p36, p37, p38, p39,
           p40, p41, p42, p43, p44, p45, p46, p47, p48, p49, p50, p51, p52,
           p53, p54, p55, p56, p57, p58, p59):
    p = [p0, p1, p2, p3, p4, p5, p6, p7, p8, p9, p10, p11, p12, p13, p14,
         p15, p16, p17, p18, p19, p20, p21, p22, p23, p24, p25, p26, p27,
         p28, p29, p30, p31, p32, p33, p34, p35, p36, p37, p38, p39, p40,
         p41, p42, p43, p44, p45, p46, p47, p48, p49, p50, p51, p52, p53,
         p54, p55, p56, p57, p58, p59]

    B, D1, D2 = inp.shape
    H, W = D2, D1
    L = 4
    hw = [(H >> i, W >> i) for i in range(L + 1)]

    downs = tuple(
        ((p[6 * i],), p[6 * i + 1], p[6 * i + 2],
         p[6 * i + 3], p[6 * i + 4], p[6 * i + 5])
        for i in range(L))
    inc = ((p[24],), p[25], p[26], p[27], p[28], p[29])
    outc = (p[30], p[31])
    ups = tuple(
        ((p[32 + 7 * i], p[33 + 7 * i]), p[34 + 7 * i], p[35 + 7 * i],
         p[36 + 7 * i], p[37 + 7 * i], p[38 + 7 * i])
        for i in range(L))

    kp = {
        "inc": inc, "downs": downs, "ups": ups, "outc": outc,
        "pool": tuple(_poolmat(_BH, *hw[i]) for i in range(L)),
        "upmat": tuple(_upmat_hilo(_BH, *hw[L - i]) for i in range(L)),
    }

    flat, treedef = jax.tree_util.tree_flatten(kp)

    stream_idx = [6, 9, 12, 15, 18, 21,
                  32, 33, 36, 39, 40, 43, 46, 47, 50, 53, 54, 57]
    hbm = {id(p[i]) for i in stream_idx}

    def _spec(a):
        nd = a.ndim
        if id(a) in hbm:
            return pl.BlockSpec(memory_space=pl.ANY)
        return pl.BlockSpec(a.shape, lambda i, _n=nd: (0,) * _n)

    x2d = jnp.transpose(inp, (0, 2, 1)).reshape(B * H, W)
    body = functools.partial(_unet_body, treedef=treedef, hw=hw, L=L,
                             n_flat=len(flat))
    y = pl.pallas_call(
        body,
        out_shape=jax.ShapeDtypeStruct((B * H, W), jnp.float32),
        grid=(B // _BH,),
        in_specs=[pl.BlockSpec((B * H, W), lambda i: (i, 0))]
        + [_spec(a) for a in flat],
        out_specs=pl.BlockSpec((B * H, W), lambda i: (i, 0)),
        scratch_shapes=[
            pltpu.VMEM(p[i].shape[1:] if i in (18, 21) else p[i].shape,
                       jnp.float32)
            for i in stream_idx
        ] + [pltpu.SemaphoreType.DMA((len(stream_idx),))],
        compiler_params=pltpu.CompilerParams(
            dimension_semantics=("arbitrary",),
            vmem_limit_bytes=64 * 1024 * 1024,
        ),
    )(x2d, *flat)

    return jnp.transpose(y.reshape(B, H, W), (0, 2, 1))


# speedup vs baseline: 1.0397x; 1.0397x over previous
import functools

import numpy as np
import jax
import jax.numpy as jnp
from jax import lax
from jax.experimental import pallas as pl
from jax.experimental.pallas import tpu as pltpu

_BH = 4
_BF = jnp.bfloat16
_F32 = jnp.float32


def _taps(H, W):
    ts = []
    for dy in range(3):
        for dx in range(3):
            oy, ox = dy - 1, dx - 1
            if (H == 1 and oy != 0) or (W == 1 and ox != 0):
                continue
            ts.append((oy, ox))
    return ts


def _interp1d(n_in, n_out):
    M = np.zeros((n_out, n_in), np.float64)
    if n_in == 1:
        M[:, 0] = 1.0
        return M
    scale = (n_in - 1) / (n_out - 1)
    for i in range(n_out):
        src = i * scale
        lo = min(int(np.floor(src)), n_in - 1)
        hi = min(lo + 1, n_in - 1)
        M[i, lo] += 1.0 - (src - lo)
        M[i, hi] += src - lo
    return M


def _upmat_hilo(Bh, H, W):
    U = np.kron(_interp1d(H, 2 * H), _interp1d(W, 2 * W))
    U = np.kron(np.eye(Bh), U).astype(np.float32)
    hi = U.astype(jnp.bfloat16.dtype)
    lo = (U - hi.astype(np.float32)).astype(jnp.bfloat16.dtype)
    return jnp.asarray(hi), jnp.asarray(lo)


def _poolmat(Bh, H, W):
    Ho, Wo = H // 2, W // 2
    S = np.zeros((Bh * Ho * Wo, Bh * H * W), np.float32)
    q = np.arange(Bh * Ho * Wo)
    b = q // (Ho * Wo)
    r = q % (Ho * Wo)
    S[q, b * H * W + 2 * (r // Wo) * W + 2 * (r % Wo)] = 1.0
    return jnp.asarray(S.astype(jnp.bfloat16.dtype))


def _shift(x, s):
    P = x.shape[0]
    s = s % P
    if s == 0:
        return x
    return jnp.concatenate([x[s:], x[:s]], axis=0)


def _split(v):
    hi = v.astype(_BF)
    lo = (v - hi.astype(_F32)).astype(_BF)
    return hi, lo


def _mk_masks(Bh, H, W):
    P = Bh * H * W
    if H == 1 and W == 1:
        return {}
    p = lax.broadcasted_iota(jnp.int32, (P, 1), 0)
    h = (p // W) % H
    w = p % W
    one = jnp.full((P, 1), 1.0, _F32)
    zero = jnp.zeros((P, 1), _F32)

    def cond1(v, o, n):
        if o == 0:
            return None
        c = (v >= 1) if o < 0 else (v <= n - 2)
        return jnp.where(c, one, zero).astype(_BF)

    out = {}
    for oy, ox in _taps(H, W):
        if (oy, ox) == (0, 0):
            continue
        mh, mw = cond1(h, oy, H), cond1(w, ox, W)
        m = mh if mw is None else (mw if mh is None else mh * mw)
        out[(oy, ox)] = m
    return out


def _conv(xs_w, sref, bref, H, W, masks):
    taps = _taps(H, W)
    acc = None
    for xv, wr in xs_w:
        xh, xl = _split(xv)
        if len(taps) == 1:
            xch, xcl = xh, xl
            wf = wr[...] if len(wr.shape) == 2 else wr[0]
        else:
            cols_h, cols_l = [], []
            for oy, ox in taps:
                sh = _shift(xh, oy * W + ox)
                sl = _shift(xl, oy * W + ox)
                if (oy, ox) != (0, 0):
                    m = masks[(oy, ox)]
                    sh = sh * m
                    sl = sl * m
                cols_h.append(sh)
                cols_l.append(sl)
            xch = jnp.concatenate(cols_h, axis=1)
            xcl = jnp.concatenate(cols_l, axis=1)
            w3 = wr[...]
            wf = w3.reshape(w3.shape[0] * w3.shape[1], w3.shape[2])
        wh, wl = _split(wf)
        d = jnp.dot(xch, wh, preferred_element_type=_F32)
        d = d + jnp.dot(xch, wl, preferred_element_type=_F32)
        d = d + jnp.dot(xcl, wh, preferred_element_type=_F32)
        acc = d if acc is None else acc + d
    y = acc * sref[...] + bref[...]
    return jnp.maximum(y, 0.0)


def _pool(x, selref, W):
    m = jnp.maximum(x, _shift(x, 1))
    m = jnp.maximum(m, _shift(m, W))
    mh, ml = _split(m)
    sel = selref[...]
    return (jnp.dot(sel, mh, preferred_element_type=_F32)
            + jnp.dot(sel, ml, preferred_element_type=_F32))


def _unet_body(*refs, treedef, hw, L, n_flat):
    x_ref = refs[0]
    p = jax.tree_util.tree_unflatten(treedef, refs[1:1 + n_flat])
    out_ref = refs[1 + n_flat]
    bufs = refs[2 + n_flat:-1]
    sems = refs[-1]
    masks = [_mk_masks(_BH, *hw[l]) for l in range(L + 1)]

    srcs = []
    for i in (1, 2, 3):
        w1 = p["downs"][i][0][0]
        w2 = p["downs"][i][3]
        if i == 3:
            srcs += [w1.at[4], w2.at[4]]
        else:
            srcs += [w1, w2]
    for i in range(L):
        (a, b), _, _, w2, _, _ = p["ups"][i]
        srcs += [a, b, w2]
    copies = [pltpu.make_async_copy(src, buf, sems.at[k])
              for k, (src, buf) in enumerate(zip(srcs, bufs))]
    for c in copies:
        c.start()

    def wait(lo, hi):
        for c in copies[lo:hi]:
            c.wait()

    def dconv(xs_w1, cp, lvl):
        _, s1, b1, w2, s2, b2 = cp
        H, W = hw[lvl]
        h = _conv(xs_w1, s1, b1, H, W, masks[lvl])
        return _conv([(h, w2)], s2, b2, H, W, masks[lvl])

    cur = dconv([(x_ref[...], p["inc"][0][0])], p["inc"], 0)
    skips = [cur]
    for i in range(L):
        pooled = _pool(cur, p["pool"][i], hw[i][1])
        cp = p["downs"][i]
        if i >= 1:
            wait(2 * (i - 1), 2 * i)
            cp = ((bufs[2 * (i - 1)],), cp[1], cp[2],
                  bufs[2 * i - 1], cp[4], cp[5])
        cur = dconv([(pooled, cp[0][0])], cp, i + 1)
        skips.append(cur)

    for i in range(L):
        uh, ul = p["upmat"][i]
        ch, cl = _split(cur)
        up = jnp.dot(uh[...], ch, preferred_element_type=_F32)
        up = up + jnp.dot(uh[...], cl, preferred_element_type=_F32)
        up = up + jnp.dot(ul[...], ch, preferred_element_type=_F32)
        wait(6 + 3 * i, 9 + 3 * i)
        w1a, w1b, w2 = bufs[6 + 3 * i], bufs[7 + 3 * i], bufs[8 + 3 * i]
        cp0 = p["ups"][i]
        cp = ((w1a, w1b), cp0[1], cp0[2], w2, cp0[4], cp0[5])
        cur = dconv([(skips[L - 1 - i], w1a), (up, w1b)],
                    cp, L - 1 - i)

    wo, bo = p["outc"]
    y = jnp.sum(cur * wo[...], axis=1, keepdims=True)
    out_ref[...] = y + bo[...]


def kernel(inp, p0, p1, p2, p3, p4, p5, p6, p7, p8, p9, p10, p11, p12, p13,
           p14, p15, p16, p17, p18, p19, p20, p21, p22, p23, p24, p25, p26,
           p27, p28, p29, p30, p31, p32, p33, p34, p35, p36, p37, p38, p39,
           p40, p41, p42, p43, p44, p45, p46, p47, p48, p49, p50, p51, p52,
           p53, p54, p55, p56, p57, p58, p59):
    p = [p0, p1, p2, p3, p4, p5, p6, p7, p8, p9, p10, p11, p12, p13, p14,
         p15, p16, p17, p18, p19, p20, p21, p22, p23, p24, p25, p26, p27,
         p28, p29, p30, p31, p32, p33, p34, p35, p36, p37, p38, p39, p40,
         p41, p42, p43, p44, p45, p46, p47, p48, p49, p50, p51, p52, p53,
         p54, p55, p56, p57, p58, p59]

    B, D1, D2 = inp.shape
    H, W = D2, D1
    L = 4
    hw = [(H >> i, W >> i) for i in range(L + 1)]

    downs = tuple(
        ((p[6 * i],), p[6 * i + 1], p[6 * i + 2],
         p[6 * i + 3], p[6 * i + 4], p[6 * i + 5])
        for i in range(L))
    inc = ((p[24],), p[25], p[26], p[27], p[28], p[29])
    outc = (p[30], p[31])
    ups = tuple(
        ((p[32 + 7 * i], p[33 + 7 * i]), p[34 + 7 * i], p[35 + 7 * i],
         p[36 + 7 * i], p[37 + 7 * i], p[38 + 7 * i])
        for i in range(L))

    kp = {
        "inc": inc, "downs": downs, "ups": ups, "outc": outc,
        "pool": tuple(_poolmat(_BH, *hw[i]) for i in range(L)),
        "upmat": tuple(_upmat_hilo(_BH, *hw[L - i]) for i in range(L)),
    }

    flat, treedef = jax.tree_util.tree_flatten(kp)

    stream_idx = [6, 9, 12, 15, 18, 21,
                  32, 33, 36, 39, 40, 43, 46, 47, 50, 53, 54, 57]
    hbm = {id(p[i]) for i in stream_idx}

    def _spec(a):
        nd = a.ndim
        if id(a) in hbm:
            return pl.BlockSpec(memory_space=pl.ANY)
        return pl.BlockSpec(a.shape, lambda i, _n=nd: (0,) * _n)

    x_pix = jnp.transpose(inp, (0, 2, 1)).reshape(B * H * W, 1)
    Pb = _BH * H * W
    body = functools.partial(_unet_body, treedef=treedef, hw=hw, L=L,
                             n_flat=len(flat))
    y = pl.pallas_call(
        body,
        out_shape=jax.ShapeDtypeStruct((B * H * W, 1), jnp.float32),
        grid=(B // _BH,),
        in_specs=[pl.BlockSpec((Pb, 1), lambda i: (i, 0))]
        + [_spec(a) for a in flat],
        out_specs=pl.BlockSpec((Pb, 1), lambda i: (i, 0)),
        scratch_shapes=[
            pltpu.VMEM(p[i].shape[1:] if i in (18, 21) else p[i].shape,
                       jnp.float32)
            for i in stream_idx
        ] + [pltpu.SemaphoreType.DMA((len(stream_idx),))],
        compiler_params=pltpu.CompilerParams(
            dimension_semantics=("arbitrary",),
            vmem_limit_bytes=64 * 1024 * 1024,
        ),
    )(x_pix, *flat)

    return jnp.transpose(y.reshape(B, H, W), (0, 2, 1))
